# baseline (speedup 1.0000x reference)
import functools

import jax
import jax.numpy as jnp
from jax import lax
from jax.experimental import pallas as pl
from jax.experimental.pallas import tpu as pltpu
from jax.experimental.pallas import tpu_sc as plsc

B = 16384
D = 128
H1, H2, OUT = 512, 256, 128

NC, NS = 2, 16
NW = NC * NS
BPW = B // NW
KCH = 128
NCH = BPW // KCH

BLK = 512
GRID = B // BLK


def _sc_gather(idx3, table):
    @functools.partial(
        pl.kernel,
        out_type=jax.ShapeDtypeStruct((B, D), jnp.float32),
        mesh=plsc.VectorSubcoreMesh(core_axis_name="c", subcore_axis_name="s"),
        scratch_types=[
            pltpu.VMEM((NCH, KCH), jnp.int32),
            pltpu.VMEM((BPW, D), jnp.float32),
            pltpu.SemaphoreType.DMA,
        ],
    )
    def k(idx_hbm, table_hbm, out_hbm, idx_v, rows_v, sem):
        wid = lax.axis_index("s") * NC + lax.axis_index("c")
        pltpu.sync_copy(idx_hbm.at[wid], idx_v)
        cps = []
        for j in range(NCH):
            cps.append(
                pltpu.async_copy(
                    table_hbm.at[idx_v.at[j]],
                    rows_v.at[pl.ds(j * KCH, KCH)],
                    sem,
                )
            )
        for cp in cps:
            cp.wait()
        pltpu.sync_copy(rows_v, out_hbm.at[pl.ds(wid * BPW, BPW)])

    return k(idx3, table)


def _mlp_body(idx_ref, emb_ref, w1_ref, b1_ref, w2_ref, b2_ref, w3_ref,
              b3_ref, o_ref):
    mask = (idx_ref[...] != 0).astype(jnp.float32)
    emb = emb_ref[...] * mask
    h = jnp.dot(emb, w1_ref[...], preferred_element_type=jnp.float32)
    h = jnp.maximum(h + b1_ref[...], 0.0)
    h = jnp.dot(h, w2_ref[...], preferred_element_type=jnp.float32)
    h = jnp.maximum(h + b2_ref[...], 0.0)
    out = jnp.dot(h, w3_ref[...], preferred_element_type=jnp.float32)
    out = out + b3_ref[...]
    ssq = jnp.sum(out * out, axis=-1, keepdims=True)
    o_ref[...] = out / jnp.maximum(jnp.sqrt(ssq), 1e-12)


def _mlp(idx2, emb, W1, b1, W2, b2, W3, b3):
    nb = idx2.shape[0]
    return pl.pallas_call(
        _mlp_body,
        grid=(nb // BLK,),
        in_specs=[
            pl.BlockSpec((BLK, 1), lambda i: (i, 0)),
            pl.BlockSpec((BLK, D), lambda i: (i, 0)),
            pl.BlockSpec((D, H1), lambda i: (0, 0)),
            pl.BlockSpec((1, H1), lambda i: (0, 0)),
            pl.BlockSpec((H1, H2), lambda i: (0, 0)),
            pl.BlockSpec((1, H2), lambda i: (0, 0)),
            pl.BlockSpec((H2, OUT), lambda i: (0, 0)),
            pl.BlockSpec((1, OUT), lambda i: (0, 0)),
        ],
        out_specs=pl.BlockSpec((BLK, OUT), lambda i: (i, 0)),
        out_shape=jax.ShapeDtypeStruct((nb, OUT), jnp.float32),
    )(idx2, emb, W1, b1, W2, b2, W3, b3)


NCHUNK = 4
BC = B // NCHUNK


def _sc_gather_chunk(idx3, table):
    kc = BC // NW

    @functools.partial(
        pl.kernel,
        out_type=jax.ShapeDtypeStruct((BC, D), jnp.float32),
        mesh=plsc.VectorSubcoreMesh(core_axis_name="c", subcore_axis_name="s"),
        scratch_types=[
            pltpu.VMEM((1, kc), jnp.int32),
            pltpu.VMEM((kc, D), jnp.float32),
            pltpu.SemaphoreType.DMA,
        ],
    )
    def k(idx_hbm, table_hbm, out_hbm, idx_v, rows_v, sem):
        wid = lax.axis_index("s") * NC + lax.axis_index("c")
        pltpu.sync_copy(idx_hbm.at[pl.ds(wid, 1)], idx_v)
        pltpu.async_copy(table_hbm.at[idx_v.at[0]], rows_v, sem).wait()
        pltpu.sync_copy(rows_v, out_hbm.at[pl.ds(wid * kc, kc)])

    return k(idx3, table)


def kernel(user_idx, table, W1, b1, W2, b2, W3, b3):
    idx = user_idx.astype(jnp.int32)
    b1r, b2r, b3r = b1.reshape(1, H1), b2.reshape(1, H2), b3.reshape(1, OUT)
    idx_c = idx.reshape(NCHUNK, NW, BC // NW)
    outs = []
    for c in range(NCHUNK):
        emb = _sc_gather_chunk(idx_c[c], table)
        outs.append(_mlp(idx_c[c].reshape(BC, 1), emb, W1, b1r, W2, b2r,
                         W3, b3r))
    return jnp.concatenate(outs, axis=0)


# speedup vs baseline: 1.2610x; 1.2610x over previous
import functools

import jax
import jax.numpy as jnp
from jax import lax
from jax.experimental import pallas as pl
from jax.experimental.pallas import tpu as pltpu
from jax.experimental.pallas import tpu_sc as plsc

B = 16384
D = 128
H1, H2, OUT = 512, 256, 128

NC, NS = 2, 16
NW = NC * NS
BPW = B // NW
KCH = 128
NCH = BPW // KCH

BLK = 512
GRID = B // BLK


def _sc_gather(idx3, table):
    @functools.partial(
        pl.kernel,
        out_type=jax.ShapeDtypeStruct((B, D), jnp.float32),
        mesh=plsc.VectorSubcoreMesh(core_axis_name="c", subcore_axis_name="s"),
        scratch_types=[
            pltpu.VMEM((NCH, KCH), jnp.int32),
            pltpu.VMEM((BPW, D), jnp.float32),
            pltpu.SemaphoreType.DMA,
        ],
    )
    def k(idx_hbm, table_hbm, out_hbm, idx_v, rows_v, sem):
        wid = lax.axis_index("s") * NC + lax.axis_index("c")
        pltpu.sync_copy(idx_hbm.at[wid], idx_v)
        cps = []
        for j in range(NCH):
            cps.append(
                pltpu.async_copy(
                    table_hbm.at[idx_v.at[j]],
                    rows_v.at[pl.ds(j * KCH, KCH)],
                    sem,
                )
            )
        for cp in cps:
            cp.wait()
        pltpu.sync_copy(rows_v, out_hbm.at[pl.ds(wid * BPW, BPW)])

    return k(idx3, table)


def _mlp_body(idx_ref, emb_ref, w1_ref, b1_ref, w2_ref, b2_ref, w3_ref,
              b3_ref, o_ref):
    mask = (idx_ref[...] != 0).astype(jnp.float32)
    emb = (emb_ref[...] * mask).astype(jnp.bfloat16)
    h = jnp.dot(emb, w1_ref[...], preferred_element_type=jnp.float32)
    h = jnp.maximum(h + b1_ref[...], 0.0).astype(jnp.bfloat16)
    h = jnp.dot(h, w2_ref[...], preferred_element_type=jnp.float32)
    h = jnp.maximum(h + b2_ref[...], 0.0).astype(jnp.bfloat16)
    out = jnp.dot(h, w3_ref[...], preferred_element_type=jnp.float32)
    out = out + b3_ref[...]
    ssq = jnp.sum(out * out, axis=-1, keepdims=True)
    o_ref[...] = out / jnp.maximum(jnp.sqrt(ssq), 1e-12)


def _mlp(idx2, emb, W1, b1, W2, b2, W3, b3):
    nb = idx2.shape[0]
    return pl.pallas_call(
        _mlp_body,
        grid=(nb // BLK,),
        in_specs=[
            pl.BlockSpec((BLK, 1), lambda i: (i, 0)),
            pl.BlockSpec((BLK, D), lambda i: (i, 0)),
            pl.BlockSpec((D, H1), lambda i: (0, 0)),
            pl.BlockSpec((1, H1), lambda i: (0, 0)),
            pl.BlockSpec((H1, H2), lambda i: (0, 0)),
            pl.BlockSpec((1, H2), lambda i: (0, 0)),
            pl.BlockSpec((H2, OUT), lambda i: (0, 0)),
            pl.BlockSpec((1, OUT), lambda i: (0, 0)),
        ],
        out_specs=pl.BlockSpec((BLK, OUT), lambda i: (i, 0)),
        out_shape=jax.ShapeDtypeStruct((nb, OUT), jnp.float32),
    )(idx2, emb, W1, b1, W2, b2, W3, b3)


NCHUNK = 4
BC = B // NCHUNK


def _sc_gather_chunk(idx3, table):
    kc = BC // NW

    @functools.partial(
        pl.kernel,
        out_type=jax.ShapeDtypeStruct((BC, D), jnp.float32),
        mesh=plsc.VectorSubcoreMesh(core_axis_name="c", subcore_axis_name="s"),
        scratch_types=[
            pltpu.VMEM((1, kc), jnp.int32),
            pltpu.VMEM((kc, D), jnp.float32),
            pltpu.SemaphoreType.DMA,
        ],
    )
    def k(idx_hbm, table_hbm, out_hbm, idx_v, rows_v, sem):
        wid = lax.axis_index("s") * NC + lax.axis_index("c")
        pltpu.sync_copy(idx_hbm.at[pl.ds(wid, 1)], idx_v)
        pltpu.async_copy(table_hbm.at[idx_v.at[0]], rows_v, sem).wait()
        pltpu.sync_copy(rows_v, out_hbm.at[pl.ds(wid * kc, kc)])

    return k(idx3, table)


def kernel(user_idx, table, W1, b1, W2, b2, W3, b3):
    idx = user_idx.astype(jnp.int32)
    emb = _sc_gather(idx.reshape(NW, NCH, KCH), table)
    return _mlp(idx.reshape(B, 1), emb,
                W1.astype(jnp.bfloat16), b1.reshape(1, H1),
                W2.astype(jnp.bfloat16), b2.reshape(1, H2),
                W3.astype(jnp.bfloat16), b3.reshape(1, OUT))


# speedup vs baseline: 1.5802x; 1.2532x over previous
import functools

import jax
import jax.numpy as jnp
from jax import lax
from jax.experimental import pallas as pl
from jax.experimental.pallas import tpu as pltpu
from jax.experimental.pallas import tpu_sc as plsc

B = 16384
D = 128
H1, H2, OUT = 512, 256, 128

NC, NS = 2, 16
NW = NC * NS
BPW = B // NW
KCH = 128
NCH = BPW // KCH

BLK = 1024
GRID = B // BLK


def _sc_gather(idx3, table):
    @functools.partial(
        pl.kernel,
        out_type=jax.ShapeDtypeStruct((B, D), jnp.float32),
        mesh=plsc.VectorSubcoreMesh(core_axis_name="c", subcore_axis_name="s"),
        scratch_types=[
            pltpu.VMEM((NCH, KCH), jnp.int32),
            pltpu.VMEM((BPW, D), jnp.float32),
            pltpu.SemaphoreType.DMA,
        ],
    )
    def k(idx_hbm, table_hbm, out_hbm, idx_v, rows_v, sem):
        wid = lax.axis_index("s") * NC + lax.axis_index("c")
        pltpu.sync_copy(idx_hbm.at[wid], idx_v)
        cps = []
        for j in range(NCH):
            cps.append(
                pltpu.async_copy(
                    table_hbm.at[idx_v.at[j]],
                    rows_v.at[pl.ds(j * KCH, KCH)],
                    sem,
                )
            )
        for cp in cps:
            cp.wait()
        pltpu.sync_copy(rows_v, out_hbm.at[pl.ds(wid * BPW, BPW)])

    return k(idx3, table)


def _mlp_body(idx_ref, emb_ref, w1_ref, b1_ref, w2_ref, b2_ref, w3_ref,
              b3_ref, o_ref):
    mask = (idx_ref[...] != 0).astype(jnp.float32)
    emb = emb_ref[...] * mask
    h = jnp.dot(emb, w1_ref[...], preferred_element_type=jnp.float32)
    h = jnp.maximum(h + b1_ref[...], 0.0)
    h = jnp.dot(h, w2_ref[...], preferred_element_type=jnp.float32)
    h = jnp.maximum(h + b2_ref[...], 0.0)
    out = jnp.dot(h, w3_ref[...], preferred_element_type=jnp.float32)
    out = out + b3_ref[...]
    ssq = jnp.sum(out * out, axis=-1, keepdims=True)
    o_ref[...] = out * jnp.minimum(lax.rsqrt(ssq), 1e12)


def _mlp(idx2, emb, W1, b1, W2, b2, W3, b3):
    nb = idx2.shape[0]
    return pl.pallas_call(
        _mlp_body,
        grid=(nb // BLK,),
        in_specs=[
            pl.BlockSpec((BLK, 1), lambda i: (i, 0)),
            pl.BlockSpec((BLK, D), lambda i: (i, 0)),
            pl.BlockSpec((D, H1), lambda i: (0, 0)),
            pl.BlockSpec((1, H1), lambda i: (0, 0)),
            pl.BlockSpec((H1, H2), lambda i: (0, 0)),
            pl.BlockSpec((1, H2), lambda i: (0, 0)),
            pl.BlockSpec((H2, OUT), lambda i: (0, 0)),
            pl.BlockSpec((1, OUT), lambda i: (0, 0)),
        ],
        out_specs=pl.BlockSpec((BLK, OUT), lambda i: (i, 0)),
        out_shape=jax.ShapeDtypeStruct((nb, OUT), jnp.float32),
    )(idx2, emb, W1, b1, W2, b2, W3, b3)


NCHUNK = 4
BC = B // NCHUNK


def _sc_gather_chunk(idx3, table):
    kc = BC // NW

    @functools.partial(
        pl.kernel,
        out_type=jax.ShapeDtypeStruct((BC, D), jnp.float32),
        mesh=plsc.VectorSubcoreMesh(core_axis_name="c", subcore_axis_name="s"),
        scratch_types=[
            pltpu.VMEM((1, kc), jnp.int32),
            pltpu.VMEM((kc, D), jnp.float32),
            pltpu.SemaphoreType.DMA,
        ],
    )
    def k(idx_hbm, table_hbm, out_hbm, idx_v, rows_v, sem):
        wid = lax.axis_index("s") * NC + lax.axis_index("c")
        pltpu.sync_copy(idx_hbm.at[pl.ds(wid, 1)], idx_v)
        pltpu.async_copy(table_hbm.at[idx_v.at[0]], rows_v, sem).wait()
        pltpu.sync_copy(rows_v, out_hbm.at[pl.ds(wid * kc, kc)])

    return k(idx3, table)


def kernel(user_idx, table, W1, b1, W2, b2, W3, b3):
    idx = user_idx.astype(jnp.int32)
    emb = _sc_gather(idx.reshape(NW, NCH, KCH), table)
    return _mlp(idx.reshape(B, 1), emb,
                W1, b1.reshape(1, H1),
                W2, b2.reshape(1, H2),
                W3, b3.reshape(1, OUT))


# speedup vs baseline: 1.7433x; 1.1032x over previous
import functools

import jax
import jax.numpy as jnp
from jax import lax
from jax.experimental import pallas as pl
from jax.experimental.pallas import tpu as pltpu
from jax.experimental.pallas import tpu_sc as plsc

B = 16384
D = 128
H1, H2, OUT = 512, 256, 128

NC, NS = 2, 16
NW = NC * NS
BPW = B // NW
KCH = 128
NCH = BPW // KCH

BLK = 2048
GRID = B // BLK


def _sc_gather(idx3, table):
    @functools.partial(
        pl.kernel,
        out_type=jax.ShapeDtypeStruct((B, D), jnp.float32),
        mesh=plsc.VectorSubcoreMesh(core_axis_name="c", subcore_axis_name="s"),
        scratch_types=[
            pltpu.VMEM((NCH, KCH), jnp.int32),
            pltpu.VMEM((BPW, D), jnp.float32),
            pltpu.SemaphoreType.DMA,
        ],
    )
    def k(idx_hbm, table_hbm, out_hbm, idx_v, rows_v, sem):
        wid = lax.axis_index("s") * NC + lax.axis_index("c")
        pltpu.sync_copy(idx_hbm.at[wid], idx_v)
        cps = []
        for j in range(NCH):
            cps.append(
                pltpu.async_copy(
                    table_hbm.at[idx_v.at[j]],
                    rows_v.at[pl.ds(j * KCH, KCH)],
                    sem,
                )
            )
        for cp in cps:
            cp.wait()
        pltpu.sync_copy(rows_v, out_hbm.at[pl.ds(wid * BPW, BPW)])

    return k(idx3, table)


def _mlp_body(idx_ref, emb_ref, w1_ref, b1_ref, w2_ref, b2_ref, w3_ref,
              b3_ref, o_ref):
    mask = (idx_ref[...] != 0).astype(jnp.float32)
    emb = emb_ref[...] * mask
    h = jnp.dot(emb, w1_ref[...], preferred_element_type=jnp.float32)
    h = jnp.maximum(h + b1_ref[...], 0.0)
    h = jnp.dot(h, w2_ref[...], preferred_element_type=jnp.float32)
    h = jnp.maximum(h + b2_ref[...], 0.0)
    out = jnp.dot(h, w3_ref[...], preferred_element_type=jnp.float32)
    out = out + b3_ref[...]
    ssq = jnp.sum(out * out, axis=-1, keepdims=True)
    o_ref[...] = out * jnp.minimum(lax.rsqrt(ssq), 1e12)


def _mlp(idx2, emb, W1, b1, W2, b2, W3, b3):
    nb = idx2.shape[0]
    return pl.pallas_call(
        _mlp_body,
        grid=(nb // BLK,),
        in_specs=[
            pl.BlockSpec((BLK, 1), lambda i: (i, 0)),
            pl.BlockSpec((BLK, D), lambda i: (i, 0)),
            pl.BlockSpec((D, H1), lambda i: (0, 0)),
            pl.BlockSpec((1, H1), lambda i: (0, 0)),
            pl.BlockSpec((H1, H2), lambda i: (0, 0)),
            pl.BlockSpec((1, H2), lambda i: (0, 0)),
            pl.BlockSpec((H2, OUT), lambda i: (0, 0)),
            pl.BlockSpec((1, OUT), lambda i: (0, 0)),
        ],
        out_specs=pl.BlockSpec((BLK, OUT), lambda i: (i, 0)),
        out_shape=jax.ShapeDtypeStruct((nb, OUT), jnp.float32),
    )(idx2, emb, W1, b1, W2, b2, W3, b3)


NCHUNK = 4
BC = B // NCHUNK


def _sc_gather_chunk(idx3, table):
    kc = BC // NW

    @functools.partial(
        pl.kernel,
        out_type=jax.ShapeDtypeStruct((BC, D), jnp.float32),
        mesh=plsc.VectorSubcoreMesh(core_axis_name="c", subcore_axis_name="s"),
        scratch_types=[
            pltpu.VMEM((1, kc), jnp.int32),
            pltpu.VMEM((kc, D), jnp.float32),
            pltpu.SemaphoreType.DMA,
        ],
    )
    def k(idx_hbm, table_hbm, out_hbm, idx_v, rows_v, sem):
        wid = lax.axis_index("s") * NC + lax.axis_index("c")
        pltpu.sync_copy(idx_hbm.at[pl.ds(wid, 1)], idx_v)
        pltpu.async_copy(table_hbm.at[idx_v.at[0]], rows_v, sem).wait()
        pltpu.sync_copy(rows_v, out_hbm.at[pl.ds(wid * kc, kc)])

    return k(idx3, table)


def kernel(user_idx, table, W1, b1, W2, b2, W3, b3):
    idx = user_idx.astype(jnp.int32)
    emb = _sc_gather(idx.reshape(NW, NCH, KCH), table)
    return _mlp(idx.reshape(B, 1), emb,
                W1, b1.reshape(1, H1),
                W2, b2.reshape(1, H2),
                W3, b3.reshape(1, OUT))


# speedup vs baseline: 1.8053x; 1.0356x over previous
import functools

import jax
import jax.numpy as jnp
from jax import lax
from jax.experimental import pallas as pl
from jax.experimental.pallas import tpu as pltpu
from jax.experimental.pallas import tpu_sc as plsc

B = 16384
D = 128
H1, H2, OUT = 512, 256, 128

NC, NS = 2, 16
NW = NC * NS
BPW = B // NW
KCH = 128
NCH = BPW // KCH

BLK = 4096
GRID = B // BLK


def _sc_gather(idx3, table):
    @functools.partial(
        pl.kernel,
        out_type=jax.ShapeDtypeStruct((B, D), jnp.float32),
        mesh=plsc.VectorSubcoreMesh(core_axis_name="c", subcore_axis_name="s"),
        scratch_types=[
            pltpu.VMEM((NCH, KCH), jnp.int32),
            pltpu.VMEM((BPW, D), jnp.float32),
            pltpu.SemaphoreType.DMA,
        ],
    )
    def k(idx_hbm, table_hbm, out_hbm, idx_v, rows_v, sem):
        wid = lax.axis_index("s") * NC + lax.axis_index("c")
        pltpu.sync_copy(idx_hbm.at[wid], idx_v)
        cps = []
        for j in range(NCH):
            cps.append(
                pltpu.async_copy(
                    table_hbm.at[idx_v.at[j]],
                    rows_v.at[pl.ds(j * KCH, KCH)],
                    sem,
                )
            )
        for cp in cps:
            cp.wait()
        pltpu.sync_copy(rows_v, out_hbm.at[pl.ds(wid * BPW, BPW)])

    return k(idx3, table)


def _mlp_body(idx_ref, emb_ref, w1_ref, b1_ref, w2_ref, b2_ref, w3_ref,
              b3_ref, o_ref):
    mask = (idx_ref[...] != 0).astype(jnp.float32)
    emb = emb_ref[...] * mask
    h = jnp.dot(emb, w1_ref[...], preferred_element_type=jnp.float32)
    h = jnp.maximum(h + b1_ref[...], 0.0)
    h = jnp.dot(h, w2_ref[...], preferred_element_type=jnp.float32)
    h = jnp.maximum(h + b2_ref[...], 0.0)
    out = jnp.dot(h, w3_ref[...], preferred_element_type=jnp.float32)
    out = out + b3_ref[...]
    ssq = jnp.sum(out * out, axis=-1, keepdims=True)
    o_ref[...] = out * jnp.minimum(lax.rsqrt(ssq), 1e12)


def _mlp(idx2, emb, W1, b1, W2, b2, W3, b3):
    nb = idx2.shape[0]
    return pl.pallas_call(
        _mlp_body,
        grid=(nb // BLK,),
        in_specs=[
            pl.BlockSpec((BLK, 1), lambda i: (i, 0)),
            pl.BlockSpec((BLK, D), lambda i: (i, 0)),
            pl.BlockSpec((D, H1), lambda i: (0, 0)),
            pl.BlockSpec((1, H1), lambda i: (0, 0)),
            pl.BlockSpec((H1, H2), lambda i: (0, 0)),
            pl.BlockSpec((1, H2), lambda i: (0, 0)),
            pl.BlockSpec((H2, OUT), lambda i: (0, 0)),
            pl.BlockSpec((1, OUT), lambda i: (0, 0)),
        ],
        out_specs=pl.BlockSpec((BLK, OUT), lambda i: (i, 0)),
        out_shape=jax.ShapeDtypeStruct((nb, OUT), jnp.float32),
    )(idx2, emb, W1, b1, W2, b2, W3, b3)


NCHUNK = 4
BC = B // NCHUNK


def _sc_gather_chunk(idx3, table):
    kc = BC // NW

    @functools.partial(
        pl.kernel,
        out_type=jax.ShapeDtypeStruct((BC, D), jnp.float32),
        mesh=plsc.VectorSubcoreMesh(core_axis_name="c", subcore_axis_name="s"),
        scratch_types=[
            pltpu.VMEM((1, kc), jnp.int32),
            pltpu.VMEM((kc, D), jnp.float32),
            pltpu.SemaphoreType.DMA,
        ],
    )
    def k(idx_hbm, table_hbm, out_hbm, idx_v, rows_v, sem):
        wid = lax.axis_index("s") * NC + lax.axis_index("c")
        pltpu.sync_copy(idx_hbm.at[pl.ds(wid, 1)], idx_v)
        pltpu.async_copy(table_hbm.at[idx_v.at[0]], rows_v, sem).wait()
        pltpu.sync_copy(rows_v, out_hbm.at[pl.ds(wid * kc, kc)])

    return k(idx3, table)


def kernel(user_idx, table, W1, b1, W2, b2, W3, b3):
    idx = user_idx.astype(jnp.int32)
    emb = _sc_gather(idx.reshape(NW, NCH, KCH), table)
    return _mlp(idx.reshape(B, 1), emb,
                W1, b1.reshape(1, H1),
                W2, b2.reshape(1, H2),
                W3, b3.reshape(1, OUT))
